# baseline (speedup 1.0000x reference)
import functools

import jax
import jax.numpy as jnp
from jax import lax
from jax.experimental import pallas as pl
from jax.experimental.pallas import tpu as pltpu
from jax.experimental.pallas import tpu_sc as plsc

_B = 4
_S = 4096
_D = 1024
_LANES = 16
_NC = 2
_NS = 16
_NW = _NC * _NS
_N = _B * _S
_PPW = _S // _NW
_CP = 8
_CR = _CP * _B
_NCH = _PPW // _CP
_NBUF = 3


def _make_kernel():
    mesh = plsc.VectorSubcoreMesh(core_axis_name="c", subcore_axis_name="s")

    @functools.partial(
        pl.kernel,
        out_type=jax.ShapeDtypeStruct((_N, _D), jnp.float32),
        mesh=mesh,
        scratch_types=[
            pltpu.VMEM((_NCH, _CR), jnp.int32),
            pltpu.VMEM((_NBUF, 2, _LANES), jnp.int32),
            pltpu.VMEM((_NBUF, _CR, _D), jnp.float32),
            pltpu.VMEM((_NBUF, _CP, _D), jnp.float32),
        ] + [pltpu.SemaphoreType.DMA] * (3 * _NBUF),
    )
    def body(ids_hbm, tok_hbm, pos_hbm, out_hbm, idx_v, oidx, tkb, psb,
             *sems):
        gs = sems[0:_NBUF]
        ps = sems[_NBUF:2 * _NBUF]
        osm = sems[2 * _NBUF:3 * _NBUF]
        wid = lax.axis_index("s") * _NC + lax.axis_index("c")
        pos0 = wid * _PPW
        pltpu.sync_copy(ids_hbm.at[wid], idx_v)
        iota = lax.iota(jnp.int32, _LANES)

        def start_g(i, b):
            pltpu.async_copy(tok_hbm.at[idx_v.at[i]], tkb.at[b], gs[b])
            pltpu.async_copy(pos_hbm.at[pl.ds(pos0 + i * _CP, _CP)],
                             psb.at[b], ps[b])

        def wait_g(b):
            pltpu.make_async_copy(tok_hbm.at[pl.ds(0, _CR)], tkb.at[b],
                                  gs[b]).wait()
            pltpu.make_async_copy(pos_hbm.at[pl.ds(0, _CP)], psb.at[b],
                                  ps[b]).wait()

        def start_o(i, b, half):
            j = iota + (half * _LANES)
            rows = ((j & (_B - 1)) * _S
                    + (j >> 2) + (pos0 + i * _CP))
            oidx[b, half, :] = rows
            pltpu.async_copy(tkb.at[b, pl.ds(half * _LANES, _LANES)],
                             out_hbm.at[oidx.at[b, half]], osm[b])

        def wait_o(b):
            for half in range(2):
                pltpu.make_async_copy(tkb.at[b, pl.ds(0, _LANES)],
                                      out_hbm.at[pl.ds(0, _LANES)],
                                      osm[b]).wait()

        def add(b, half):
            def prow(p, c2):
                for c in range(_D // _LANES):
                    sl = pl.ds(c * _LANES, _LANES)
                    vpos = psb[b, p, sl]
                    for bb in range(_B):
                        plsc.addupdate(tkb.at[b, p * _B + bb, sl], vpos)
                return c2

            lax.fori_loop(half * (_CP // 2), (half + 1) * (_CP // 2), prow, 0)

        def step(i, b, b2, first, last):
            wait_g(b)
            add(b, 0)
            start_o(i, b, 0)
            add(b, 1)
            start_o(i, b, 1)
            if not last:
                if not first:
                    wait_o(b2)
                start_g(i + 2, b2)

        start_g(0, 0)
        start_g(1, 1)
        step(0, 0, 2, True, False)
        step(1, 1, 0, False, False)

        def group(g, carry):
            for j in range(_NBUF):
                i = 2 + g * _NBUF + j
                step(i, (2 + j) % _NBUF, (4 + j) % _NBUF, False, False)
            return carry

        lax.fori_loop(0, (_NCH - 4) // _NBUF, group, 0)

        step(_NCH - 2, (_NCH - 2) % _NBUF, 0, False, True)
        step(_NCH - 1, (_NCH - 1) % _NBUF, 0, False, True)
        for k in range(_NCH - 3, _NCH):
            wait_o(k % _NBUF)

    return body


_kernel_fn = _make_kernel()


def kernel(input_ids, token_table, pos_table):
    ids = jnp.transpose(input_ids.astype(jnp.int32)).reshape(_NW, _NCH, _CR)
    out = _kernel_fn(ids, token_table, pos_table)
    return out.reshape(_B, _S, _D)


# speedup vs baseline: 1.0341x; 1.0341x over previous
import functools

import jax
import jax.numpy as jnp
from jax import lax
from jax.experimental import pallas as pl
from jax.experimental.pallas import tpu as pltpu
from jax.experimental.pallas import tpu_sc as plsc

_B = 4
_S = 4096
_D = 1024
_LANES = 16
_NC = 2
_NS = 16
_NW = _NC * _NS
_N = _B * _S
_PPW = _S // _NW
_CP = 8
_CR = _CP * _B
_NCH = _PPW // _CP
_NBUF = 3


def _make_kernel():
    mesh = plsc.VectorSubcoreMesh(core_axis_name="c", subcore_axis_name="s")

    @functools.partial(
        pl.kernel,
        out_type=jax.ShapeDtypeStruct((_N, _D), jnp.float32),
        mesh=mesh,
        scratch_types=[
            pltpu.VMEM((_NCH, _CR), jnp.int32),
            pltpu.VMEM((_NBUF, _CR), jnp.int32),
            pltpu.VMEM((_NBUF, _CR, _D), jnp.float32),
            pltpu.VMEM((_NBUF, _CP, _D), jnp.float32),
        ] + [pltpu.SemaphoreType.DMA] * (3 * _NBUF),
    )
    def body(ids_hbm, tok_hbm, pos_hbm, out_hbm, idx_v, oidx, tkb, psb,
             *sems):
        gs = sems[0:_NBUF]
        ps = sems[_NBUF:2 * _NBUF]
        osm = sems[2 * _NBUF:3 * _NBUF]
        wid = lax.axis_index("s") * _NC + lax.axis_index("c")
        pos0 = wid * _PPW
        pltpu.sync_copy(ids_hbm.at[wid], idx_v)
        iota = lax.iota(jnp.int32, _LANES)

        def start_g(i, b):
            pltpu.async_copy(tok_hbm.at[idx_v.at[i]], tkb.at[b], gs[b])
            pltpu.async_copy(pos_hbm.at[pl.ds(pos0 + i * _CP, _CP)],
                             psb.at[b], ps[b])

        def wait_g(b):
            pltpu.make_async_copy(tok_hbm.at[pl.ds(0, _CR)], tkb.at[b],
                                  gs[b]).wait()
            pltpu.make_async_copy(pos_hbm.at[pl.ds(0, _CP)], psb.at[b],
                                  ps[b]).wait()

        def start_o(i, b):
            for h in range(_CR // _LANES):
                j = iota + (h * _LANES)
                rows = ((j & (_B - 1)) * _S
                        + (j >> 2) + (pos0 + i * _CP))
                oidx[b, pl.ds(h * _LANES, _LANES)] = rows
            pltpu.async_copy(tkb.at[b], out_hbm.at[oidx.at[b]], osm[b])

        def wait_o(b):
            pltpu.make_async_copy(tkb.at[b], out_hbm.at[pl.ds(0, _CR)],
                                  osm[b]).wait()

        def add(b):
            def prow(p, c2):
                for c in range(_D // _LANES):
                    sl = pl.ds(c * _LANES, _LANES)
                    vpos = psb[b, p, sl]
                    for bb in range(_B):
                        plsc.addupdate(tkb.at[b, p * _B + bb, sl], vpos)
                return c2

            lax.fori_loop(0, _CP, prow, 0)

        def step(i, b, b2, first, last):
            wait_g(b)
            add(b)
            if not last:
                if not first:
                    wait_o(b2)
                start_g(i + 2, b2)
            start_o(i, b)

        start_g(0, 0)
        start_g(1, 1)
        step(0, 0, 2, True, False)
        step(1, 1, 0, False, False)

        def group(g, carry):
            for j in range(_NBUF):
                i = 2 + g * _NBUF + j
                step(i, (2 + j) % _NBUF, (4 + j) % _NBUF, False, False)
            return carry

        lax.fori_loop(0, (_NCH - 4) // _NBUF, group, 0)

        step(_NCH - 2, (_NCH - 2) % _NBUF, 0, False, True)
        step(_NCH - 1, (_NCH - 1) % _NBUF, 0, False, True)
        for k in range(_NCH - 3, _NCH):
            wait_o(k % _NBUF)

    return body


_kernel_fn = _make_kernel()


def kernel(input_ids, token_table, pos_table):
    ids = jnp.transpose(input_ids.astype(jnp.int32)).reshape(_NW, _NCH, _CR)
    out = _kernel_fn(ids, token_table, pos_table)
    return out.reshape(_B, _S, _D)
